# baseline (speedup 1.0000x reference)
import functools

import jax
import jax.numpy as jnp
from jax import lax
from jax.experimental import pallas as pl
from jax.experimental.pallas import tpu as pltpu
from jax.experimental.pallas import tpu_sc as plsc

NUM_EXPERTS = 8
TOP_K = 2
HIDDEN = 1024
INTER = 2048
TOKENS = 2048

NW = 32
TPW = TOKENS // NW
B_R = 128
NB = 32 + NUM_EXPERTS - 1
N_SLOTS = 5120
SPW = N_SLOTS // NW
NEG_INF = float("-inf")

_mesh = plsc.VectorSubcoreMesh(core_axis_name="c", subcore_axis_name="s")


def _wid():
    return lax.axis_index("c") * 16 + lax.axis_index("s")


def _lane():
    return lax.iota(jnp.int32, 16)


def _bsum(x):
    cs = plsc.cumsum(x)
    return plsc.cummax(lax.rev(cs, (0,)))


def _pick(vec, lane_idx):
    return _bsum(jnp.where(_lane() == lane_idx, vec, jnp.zeros_like(vec)))


def _a1_body(rl_hbm, ids_hbm, w_hbm, counts_hbm, rl_v, ids_v, w_v, cnt_v):
    wid = _wid()
    base = wid * TPW
    pltpu.sync_copy(rl_hbm, rl_v)

    cnt = [jnp.zeros((16,), jnp.int32) for _ in range(NUM_EXPERTS)]
    for c in range(TPW // 16):
        ls = [rl_v[e, pl.ds(base + 16 * c, 16)] for e in range(NUM_EXPERTS)]
        best_v = ls[0]
        best_i = jnp.zeros((16,), jnp.int32)
        for e in range(1, NUM_EXPERTS):
            m = ls[e] > best_v
            best_i = jnp.where(m, e, best_i)
            best_v = jnp.where(m, ls[e], best_v)
        sec_v = jnp.full((16,), NEG_INF, jnp.float32)
        sec_i = jnp.zeros((16,), jnp.int32)
        for e in range(NUM_EXPERTS):
            cand = jnp.where(best_i == e, NEG_INF, ls[e])
            m = cand > sec_v
            sec_i = jnp.where(m, e, sec_i)
            sec_v = jnp.where(m, cand, sec_v)
        d = best_v - sec_v
        w2nd = 1.0 / (1.0 + jnp.exp(d))
        w1st = 1.0 - w2nd
        ids_v[0, pl.ds(16 * c, 16)] = best_i
        ids_v[1, pl.ds(16 * c, 16)] = sec_i
        w_v[0, pl.ds(16 * c, 16)] = w1st
        w_v[1, pl.ds(16 * c, 16)] = w2nd
        for e in range(NUM_EXPERTS):
            cnt[e] = cnt[e] + jnp.where(best_i == e, 1, 0)
            cnt[e] = cnt[e] + jnp.where(sec_i == e, 1, 0)

    lane = _lane()
    cvec = jnp.zeros((16,), jnp.int32)
    for e in range(NUM_EXPERTS):
        cvec = jnp.where(lane == e, _bsum(cnt[e]), cvec)
    cnt_v[...] = cvec

    for k in range(TOP_K):
        pltpu.sync_copy(ids_v.at[k], ids_hbm.at[k, pl.ds(base, TPW)])
        pltpu.sync_copy(w_v.at[k], w_hbm.at[k, pl.ds(base, TPW)])
    pltpu.sync_copy(cnt_v, counts_hbm.at[wid])


_a1 = functools.partial(
    pl.kernel,
    out_type=(
        jax.ShapeDtypeStruct((TOP_K, TOKENS), jnp.int32),
        jax.ShapeDtypeStruct((TOP_K, TOKENS), jnp.float32),
        jax.ShapeDtypeStruct((NW, 16), jnp.int32),
    ),
    mesh=_mesh,
    compiler_params=pltpu.CompilerParams(needs_layout_passes=False),
    scratch_types=[
        pltpu.VMEM((NUM_EXPERTS, TOKENS), jnp.float32),
        pltpu.VMEM((TOP_K, TPW), jnp.int32),
        pltpu.VMEM((TOP_K, TPW), jnp.float32),
        pltpu.VMEM((16,), jnp.int32),
    ],
)(_a1_body)


def _global_counts(cnt_v, wid=None):
    total_v = jnp.zeros((16,), jnp.int32)
    for r in range(NW):
        total_v = total_v + cnt_v[r, :]
    s = [_pick(total_v, e) for e in range(NUM_EXPERTS)]
    if wid is None:
        return s, None
    pref_v = lax.fori_loop(0, wid, lambda r, a: a + cnt_v[r, :],
                           jnp.zeros((16,), jnp.int32))
    p = [_pick(pref_v, e) for e in range(NUM_EXPERTS)]
    return s, p


def _block_starts(s):
    bs, cum = [], []
    run_rows = jnp.zeros((16,), jnp.int32)
    run_blocks = jnp.zeros((16,), jnp.int32)
    for e in range(NUM_EXPERTS):
        bs.append(run_rows)
        nb_e = (s[e] + (B_R - 1)) // B_R
        run_blocks = run_blocks + nb_e
        run_rows = run_rows + nb_e * B_R
        cum.append(run_blocks)
    return bs, cum


def _a2_body(ids_hbm, counts_hbm, tokrows_hbm,
             pos_hbm, meta_hbm, bmeta_hbm,
             cnt_v, ids_v, idx0_v, idx1_v, tok_v, bm_v):
    wid = _wid()
    base = wid * TPW
    pltpu.sync_copy(counts_hbm, cnt_v)
    for k in range(TOP_K):
        pltpu.sync_copy(ids_hbm.at[k, pl.ds(base, TPW)], ids_v.at[k])

    s, p = _global_counts(cnt_v, wid)
    bs, cum = _block_starts(s)
    my_off = [bs[e] + p[e] for e in range(NUM_EXPERTS)]

    for k in range(TOP_K):
        idx_ref = idx0_v if k == 0 else idx1_v
        for c in range(TPW // 16):
            idv = ids_v[k, pl.ds(16 * c, 16)]
            posv = jnp.zeros((16,), jnp.int32)
            for e in range(NUM_EXPERTS):
                m = idv == e
                mi = jnp.where(m, 1, 0)
                csum = plsc.cumsum(mi)
                cand = (my_off[e] - 1) + csum
                posv = jnp.where(m, cand, posv)
                my_off[e] = my_off[e] + plsc.cummax(lax.rev(csum, (0,)))
            idx_ref[pl.ds(16 * c, 16)] = posv

    pltpu.sync_copy(idx0_v, pos_hbm.at[0, pl.ds(base, TPW)])
    pltpu.sync_copy(idx1_v, pos_hbm.at[1, pl.ds(base, TPW)])

    pltpu.sync_copy(tokrows_hbm.at[pl.ds(base, TPW)], tok_v)
    pltpu.sync_copy(tok_v, meta_hbm.at[idx0_v])
    pltpu.sync_copy(tok_v, meta_hbm.at[idx1_v])

    @pl.when(wid == 0)
    def _():
        lane = _lane()
        for v in range(3):
            bvec = lane + 16 * v
            acc = jnp.zeros((16,), jnp.int32)
            for e in range(NUM_EXPERTS):
                acc = acc + jnp.where(bvec >= cum[e], 1, 0)
            bexp = jnp.minimum(acc, NUM_EXPERTS - 1)
            if v == 2:
                bexp = jnp.where(lane == NB - 32, cum[NUM_EXPERTS - 1], bexp)
            bm_v[pl.ds(16 * v, 16)] = bexp
        pltpu.sync_copy(bm_v, bmeta_hbm)


_a2 = functools.partial(
    pl.kernel,
    out_type=(
        jax.ShapeDtypeStruct((TOP_K, TOKENS), jnp.int32),
        jax.ShapeDtypeStruct((N_SLOTS, 128), jnp.int32),
        jax.ShapeDtypeStruct((48,), jnp.int32),
    ),
    mesh=_mesh,
    compiler_params=pltpu.CompilerParams(needs_layout_passes=False),
    scratch_types=[
        pltpu.VMEM((NW, 16), jnp.int32),
        pltpu.VMEM((TOP_K, TPW), jnp.int32),
        pltpu.VMEM((TPW,), jnp.int32),
        pltpu.VMEM((TPW,), jnp.int32),
        pltpu.VMEM((TPW, 128), jnp.int32),
        pltpu.VMEM((48,), jnp.int32),
    ],
)(_a2_body)


def _a3_body(meta_hbm, counts_hbm, x_hbm, xs_hbm,
             cnt_v, mrows_v, idx_v, buf0_v, buf1_v, buf2_v,
             sem0, sem1, sem2):
    wid = _wid()
    mybase = wid * SPW
    pltpu.sync_copy(counts_hbm, cnt_v)
    pltpu.sync_copy(meta_hbm.at[pl.ds(mybase, SPW)], mrows_v)

    s, _ = _global_counts(cnt_v)
    bs, _ = _block_starts(s)

    lane = _lane()
    for v in range(SPW // 16):
        ids16 = jnp.zeros((16,), jnp.int32)
        for r in range(16):
            ids16 = ids16 + jnp.where(
                lane == r, mrows_v[16 * v + r, pl.ds(0, 16)], 0)
        slot = mybase + 16 * v + lane
        valid = jnp.zeros((16,), jnp.bool_)
        for e in range(NUM_EXPERTS):
            valid = jnp.logical_or(
                valid,
                jnp.logical_and(slot >= bs[e], slot < bs[e] + s[e]))
        idx_v[pl.ds(16 * v, 16)] = jnp.where(valid, ids16, 0)

    NR = 5
    RR = SPW // NR
    bufs = [buf0_v, buf1_v, buf2_v]
    sems = [sem0, sem1, sem2]
    cps = [None] * NR

    def start(rr):
        cps[rr] = pltpu.async_copy(
            x_hbm.at[idx_v.at[pl.ds(RR * rr, RR)]],
            bufs[rr % 3], sems[rr % 3])

    start(0)
    start(1)
    for rr in range(NR):
        cps[rr].wait()
        if rr + 2 < NR:
            start(rr + 2)
        pltpu.sync_copy(bufs[rr % 3], xs_hbm.at[pl.ds(mybase + RR * rr, RR)])


_a3 = functools.partial(
    pl.kernel,
    out_type=jax.ShapeDtypeStruct((N_SLOTS, HIDDEN // 2), jnp.int32),
    mesh=_mesh,
    compiler_params=pltpu.CompilerParams(needs_layout_passes=False),
    scratch_types=[
        pltpu.VMEM((NW, 16), jnp.int32),
        pltpu.VMEM((SPW, 128), jnp.int32),
        pltpu.VMEM((SPW,), jnp.int32),
        pltpu.VMEM((SPW // 5, HIDDEN // 2), jnp.int32),
        pltpu.VMEM((SPW // 5, HIDDEN // 2), jnp.int32),
        pltpu.VMEM((SPW // 5, HIDDEN // 2), jnp.int32),
        pltpu.SemaphoreType.DMA,
        pltpu.SemaphoreType.DMA,
        pltpu.SemaphoreType.DMA,
    ],
)(_a3_body)


def _b_body(bm_ref, xs_ref, w13_ref, w2_ref, y_ref):
    b = pl.program_id(0)
    nused = bm_ref[NB]

    @pl.when(b < nused)
    def _():
        xb = xs_ref[...]
        gu = lax.dot_general(xb, w13_ref[0], (((1,), (0,)), ((), ())),
                             preferred_element_type=jnp.float32)
        gate = gu[:, :INTER]
        up = gu[:, INTER:]
        h = ((gate * jax.nn.sigmoid(gate)) * up).astype(jnp.bfloat16)
        y_ref[...] = lax.dot_general(h, w2_ref[0], (((1,), (0,)), ((), ())),
                                     preferred_element_type=jnp.float32)


def _b_call(bmeta, xs2, w13t, w2t):
    def eff(b, sref):
        return jnp.minimum(b, sref[NB] - 1)

    grid_spec = pltpu.PrefetchScalarGridSpec(
        num_scalar_prefetch=1,
        grid=(NB,),
        in_specs=[
            pl.BlockSpec((B_R, HIDDEN), lambda b, sref: (eff(b, sref), 0)),
            pl.BlockSpec((1, HIDDEN, 2 * INTER),
                         lambda b, sref: (sref[eff(b, sref)], 0, 0)),
            pl.BlockSpec((1, INTER, HIDDEN),
                         lambda b, sref: (sref[eff(b, sref)], 0, 0)),
        ],
        out_specs=pl.BlockSpec((B_R, HIDDEN),
                               lambda b, sref: (eff(b, sref), 0)),
    )
    return pl.pallas_call(
        _b_body,
        grid_spec=grid_spec,
        out_shape=jax.ShapeDtypeStruct((NB * B_R, HIDDEN), jnp.float32),
        compiler_params=pltpu.CompilerParams(
            dimension_semantics=("arbitrary",),
        ),
    )(bmeta, xs2, w13t, w2t)


def _c_body(y_hbm, pos_hbm, w_hbm, out_hbm,
            idx0_v, idx1_v, w0_v, w1_v, y0_v, y1_v, sem_a, sem_b):
    wid = _wid()
    lane = _lane()
    CH = 32
    for cc in range(TPW // CH):
        tokbase = wid * TPW + cc * CH
        pltpu.sync_copy(pos_hbm.at[0, pl.ds(tokbase, CH)], idx0_v)
        pltpu.sync_copy(pos_hbm.at[1, pl.ds(tokbase, CH)], idx1_v)
        pltpu.sync_copy(w_hbm.at[0, pl.ds(tokbase, CH)], w0_v)
        pltpu.sync_copy(w_hbm.at[1, pl.ds(tokbase, CH)], w1_v)
        cp_a = pltpu.async_copy(y_hbm.at[idx0_v], y0_v, sem_a)
        cp_b = pltpu.async_copy(y_hbm.at[idx1_v], y1_v, sem_b)
        cp_a.wait()
        cp_b.wait()

        wlo0, whi0 = w0_v[pl.ds(0, 16)], w0_v[pl.ds(16, 16)]
        wlo1, whi1 = w1_v[pl.ds(0, 16)], w1_v[pl.ds(16, 16)]

        def row_fn(r, _):
            z = jnp.zeros((16,), jnp.float32)
            w0s = (_bsum(jnp.where(lane == r, wlo0, z))
                   + _bsum(jnp.where(lane == r - 16, whi0, z)))
            w1s = (_bsum(jnp.where(lane == r, wlo1, z))
                   + _bsum(jnp.where(lane == r - 16, whi1, z)))
            for j in range(HIDDEN // 16):
                sl = pl.ds(16 * j, 16)
                y0_v[r, sl] = y0_v[r, sl] * w0s + y1_v[r, sl] * w1s
            return 0

        lax.fori_loop(0, CH, row_fn, 0)
        pltpu.sync_copy(y0_v, out_hbm.at[pl.ds(tokbase, CH)])


_c = functools.partial(
    pl.kernel,
    out_type=jax.ShapeDtypeStruct((TOKENS, HIDDEN), jnp.float32),
    mesh=_mesh,
    compiler_params=pltpu.CompilerParams(needs_layout_passes=False),
    scratch_types=[
        pltpu.VMEM((32,), jnp.int32),
        pltpu.VMEM((32,), jnp.int32),
        pltpu.VMEM((32,), jnp.float32),
        pltpu.VMEM((32,), jnp.float32),
        pltpu.VMEM((32, HIDDEN), jnp.float32),
        pltpu.VMEM((32, HIDDEN), jnp.float32),
        pltpu.SemaphoreType.DMA,
        pltpu.SemaphoreType.DMA,
    ],
)(_c_body)


@jax.jit
def kernel(x, router_logits, w13_weight, w2_weight):
    rlt = router_logits.T
    w13t = jnp.transpose(w13_weight, (0, 2, 1)).astype(jnp.bfloat16)
    w2t = jnp.transpose(w2_weight, (0, 2, 1)).astype(jnp.bfloat16)
    tokrows = jnp.broadcast_to(
        jnp.arange(TOKENS, dtype=jnp.int32)[:, None], (TOKENS, 128))

    ids, wts, counts = _a1(rlt)
    pos, meta, bmeta = _a2(ids, counts, tokrows)
    x_pk = lax.bitcast_convert_type(
        x.astype(jnp.bfloat16).reshape(TOKENS, HIDDEN // 2, 2), jnp.int32)
    xs_pk = _a3(meta, counts, x_pk)
    xs_bf = lax.bitcast_convert_type(
        xs_pk, jnp.bfloat16).reshape(N_SLOTS, HIDDEN)
    y = _b_call(bmeta, xs_bf, w13t, w2t)
    out = _c(y, pos, wts)
    return out


# speedup vs baseline: 1.7096x; 1.7096x over previous
import functools

import jax
import jax.numpy as jnp
from jax import lax
from jax.experimental import pallas as pl
from jax.experimental.pallas import tpu as pltpu
from jax.experimental.pallas import tpu_sc as plsc

NUM_EXPERTS = 8
TOP_K = 2
HIDDEN = 1024
INTER = 2048
TOKENS = 2048

NW = 32
TPW = TOKENS // NW
B_R = 128
NB = 32 + NUM_EXPERTS - 1
N_SLOTS = 5120
SPW = N_SLOTS // NW
NEG_INF = float("-inf")

_mesh = plsc.VectorSubcoreMesh(core_axis_name="c", subcore_axis_name="s")


def _wid():
    return lax.axis_index("c") * 16 + lax.axis_index("s")


def _lane():
    return lax.iota(jnp.int32, 16)


def _bsum(x):
    cs = plsc.cumsum(x)
    return plsc.cummax(lax.rev(cs, (0,)))


def _pick(vec, lane_idx):
    return _bsum(jnp.where(_lane() == lane_idx, vec, jnp.zeros_like(vec)))


def _a1_body(rl_hbm, ids_hbm, w_hbm, counts_hbm, rl_v, ids_v, w_v, cnt_v):
    wid = _wid()
    base = wid * TPW
    pltpu.sync_copy(rl_hbm, rl_v)

    cnt = [jnp.zeros((16,), jnp.int32) for _ in range(NUM_EXPERTS)]
    for c in range(TPW // 16):
        ls = [rl_v[e, pl.ds(base + 16 * c, 16)] for e in range(NUM_EXPERTS)]
        best_v = ls[0]
        best_i = jnp.zeros((16,), jnp.int32)
        for e in range(1, NUM_EXPERTS):
            m = ls[e] > best_v
            best_i = jnp.where(m, e, best_i)
            best_v = jnp.where(m, ls[e], best_v)
        sec_v = jnp.full((16,), NEG_INF, jnp.float32)
        sec_i = jnp.zeros((16,), jnp.int32)
        for e in range(NUM_EXPERTS):
            cand = jnp.where(best_i == e, NEG_INF, ls[e])
            m = cand > sec_v
            sec_i = jnp.where(m, e, sec_i)
            sec_v = jnp.where(m, cand, sec_v)
        d = best_v - sec_v
        w2nd = 1.0 / (1.0 + jnp.exp(d))
        w1st = 1.0 - w2nd
        ids_v[0, pl.ds(16 * c, 16)] = best_i
        ids_v[1, pl.ds(16 * c, 16)] = sec_i
        w_v[0, pl.ds(16 * c, 16)] = w1st
        w_v[1, pl.ds(16 * c, 16)] = w2nd
        for e in range(NUM_EXPERTS):
            cnt[e] = cnt[e] + jnp.where(best_i == e, 1, 0)
            cnt[e] = cnt[e] + jnp.where(sec_i == e, 1, 0)

    lane = _lane()
    cvec = jnp.zeros((16,), jnp.int32)
    for e in range(NUM_EXPERTS):
        cvec = jnp.where(lane == e, _bsum(cnt[e]), cvec)
    cnt_v[...] = cvec

    for k in range(TOP_K):
        pltpu.sync_copy(ids_v.at[k], ids_hbm.at[k, pl.ds(base, TPW)])
        pltpu.sync_copy(w_v.at[k], w_hbm.at[k, pl.ds(base, TPW)])
    pltpu.sync_copy(cnt_v, counts_hbm.at[wid])


_a1 = functools.partial(
    pl.kernel,
    out_type=(
        jax.ShapeDtypeStruct((TOP_K, TOKENS), jnp.int32),
        jax.ShapeDtypeStruct((TOP_K, TOKENS), jnp.float32),
        jax.ShapeDtypeStruct((NW, 16), jnp.int32),
    ),
    mesh=_mesh,
    compiler_params=pltpu.CompilerParams(needs_layout_passes=False),
    scratch_types=[
        pltpu.VMEM((NUM_EXPERTS, TOKENS), jnp.float32),
        pltpu.VMEM((TOP_K, TPW), jnp.int32),
        pltpu.VMEM((TOP_K, TPW), jnp.float32),
        pltpu.VMEM((16,), jnp.int32),
    ],
)(_a1_body)


def _global_counts(cnt_v, wid=None):
    total_v = jnp.zeros((16,), jnp.int32)
    for r in range(NW):
        total_v = total_v + cnt_v[r, :]
    s = [_pick(total_v, e) for e in range(NUM_EXPERTS)]
    if wid is None:
        return s, None
    pref_v = lax.fori_loop(0, wid, lambda r, a: a + cnt_v[r, :],
                           jnp.zeros((16,), jnp.int32))
    p = [_pick(pref_v, e) for e in range(NUM_EXPERTS)]
    return s, p


def _block_starts(s):
    bs, cum = [], []
    run_rows = jnp.zeros((16,), jnp.int32)
    run_blocks = jnp.zeros((16,), jnp.int32)
    for e in range(NUM_EXPERTS):
        bs.append(run_rows)
        nb_e = (s[e] + (B_R - 1)) // B_R
        run_blocks = run_blocks + nb_e
        run_rows = run_rows + nb_e * B_R
        cum.append(run_blocks)
    return bs, cum


def _a2_body(ids_hbm, counts_hbm, x_hbm,
             pos_hbm, bmeta_hbm, xs_hbm,
             cnt_v, ids_v, idx0_v, idx1_v, xrows_v, bm_v):
    wid = _wid()
    base = wid * TPW
    pltpu.sync_copy(counts_hbm, cnt_v)
    for k in range(TOP_K):
        pltpu.sync_copy(ids_hbm.at[k, pl.ds(base, TPW)], ids_v.at[k])

    s, p = _global_counts(cnt_v, wid)
    bs, cum = _block_starts(s)
    my_off = [bs[e] + p[e] for e in range(NUM_EXPERTS)]

    for k in range(TOP_K):
        idx_ref = idx0_v if k == 0 else idx1_v
        for c in range(TPW // 16):
            idv = ids_v[k, pl.ds(16 * c, 16)]
            posv = jnp.zeros((16,), jnp.int32)
            for e in range(NUM_EXPERTS):
                m = idv == e
                mi = jnp.where(m, 1, 0)
                csum = plsc.cumsum(mi)
                cand = (my_off[e] - 1) + csum
                posv = jnp.where(m, cand, posv)
                my_off[e] = my_off[e] + plsc.cummax(lax.rev(csum, (0,)))
            idx_ref[pl.ds(16 * c, 16)] = posv

    pltpu.sync_copy(idx0_v, pos_hbm.at[0, pl.ds(base, TPW)])
    pltpu.sync_copy(idx1_v, pos_hbm.at[1, pl.ds(base, TPW)])

    pltpu.sync_copy(x_hbm.at[pl.ds(base, TPW)], xrows_v)
    pltpu.sync_copy(xrows_v, xs_hbm.at[idx0_v])
    pltpu.sync_copy(xrows_v, xs_hbm.at[idx1_v])

    @pl.when(wid == 0)
    def _():
        lane = _lane()
        for v in range(3):
            bvec = lane + 16 * v
            acc = jnp.zeros((16,), jnp.int32)
            for e in range(NUM_EXPERTS):
                acc = acc + jnp.where(bvec >= cum[e], 1, 0)
            bexp = jnp.minimum(acc, NUM_EXPERTS - 1)
            if v == 2:
                bexp = jnp.where(lane == NB - 32, cum[NUM_EXPERTS - 1], bexp)
            bm_v[pl.ds(16 * v, 16)] = bexp
        pltpu.sync_copy(bm_v, bmeta_hbm)


_a2 = functools.partial(
    pl.kernel,
    out_type=(
        jax.ShapeDtypeStruct((TOP_K, TOKENS), jnp.int32),
        jax.ShapeDtypeStruct((48,), jnp.int32),
        jax.ShapeDtypeStruct((N_SLOTS, HIDDEN), jnp.float32),
    ),
    mesh=_mesh,
    compiler_params=pltpu.CompilerParams(needs_layout_passes=False),
    scratch_types=[
        pltpu.VMEM((NW, 16), jnp.int32),
        pltpu.VMEM((TOP_K, TPW), jnp.int32),
        pltpu.VMEM((TPW,), jnp.int32),
        pltpu.VMEM((TPW,), jnp.int32),
        pltpu.VMEM((TPW, HIDDEN), jnp.float32),
        pltpu.VMEM((48,), jnp.int32),
    ],
)(_a2_body)


def _b_body(bm_ref, xs_ref, w13_ref, w2_ref, y_ref):
    b = pl.program_id(0)
    nused = bm_ref[NB]

    @pl.when(b < nused)
    def _():
        xb = xs_ref[...].astype(jnp.bfloat16)
        gu = lax.dot_general(xb, w13_ref[0], (((1,), (0,)), ((), ())),
                             preferred_element_type=jnp.float32)
        gate = gu[:, :INTER]
        up = gu[:, INTER:]
        h = ((gate * jax.nn.sigmoid(gate)) * up).astype(jnp.bfloat16)
        y_ref[...] = lax.dot_general(h, w2_ref[0], (((1,), (0,)), ((), ())),
                                     preferred_element_type=jnp.float32)


def _b_call(bmeta, xs2, w13t, w2t):
    def eff(b, sref):
        return jnp.minimum(b, sref[NB] - 1)

    grid_spec = pltpu.PrefetchScalarGridSpec(
        num_scalar_prefetch=1,
        grid=(NB,),
        in_specs=[
            pl.BlockSpec((B_R, HIDDEN), lambda b, sref: (eff(b, sref), 0)),
            pl.BlockSpec((1, HIDDEN, 2 * INTER),
                         lambda b, sref: (sref[eff(b, sref)], 0, 0)),
            pl.BlockSpec((1, INTER, HIDDEN),
                         lambda b, sref: (sref[eff(b, sref)], 0, 0)),
        ],
        out_specs=pl.BlockSpec((B_R, HIDDEN),
                               lambda b, sref: (eff(b, sref), 0)),
    )
    return pl.pallas_call(
        _b_body,
        grid_spec=grid_spec,
        out_shape=jax.ShapeDtypeStruct((NB * B_R, HIDDEN), jnp.float32),
        compiler_params=pltpu.CompilerParams(
            dimension_semantics=("arbitrary",),
        ),
    )(bmeta, xs2, w13t, w2t)


def _c_body(y_hbm, pos_hbm, w_hbm, out_hbm,
            idx0_v, idx1_v, w0_v, w1_v, y0_v, y1_v, sem_a, sem_b):
    wid = _wid()
    lane = _lane()
    CH = 32
    for cc in range(TPW // CH):
        tokbase = wid * TPW + cc * CH
        pltpu.sync_copy(pos_hbm.at[0, pl.ds(tokbase, CH)], idx0_v)
        pltpu.sync_copy(pos_hbm.at[1, pl.ds(tokbase, CH)], idx1_v)
        pltpu.sync_copy(w_hbm.at[0, pl.ds(tokbase, CH)], w0_v)
        pltpu.sync_copy(w_hbm.at[1, pl.ds(tokbase, CH)], w1_v)
        cp_a = pltpu.async_copy(y_hbm.at[idx0_v], y0_v, sem_a)
        cp_b = pltpu.async_copy(y_hbm.at[idx1_v], y1_v, sem_b)
        cp_a.wait()
        cp_b.wait()

        wlo0, whi0 = w0_v[pl.ds(0, 16)], w0_v[pl.ds(16, 16)]
        wlo1, whi1 = w1_v[pl.ds(0, 16)], w1_v[pl.ds(16, 16)]

        def row_fn(r, _):
            z = jnp.zeros((16,), jnp.float32)
            w0s = (_bsum(jnp.where(lane == r, wlo0, z))
                   + _bsum(jnp.where(lane == r - 16, whi0, z)))
            w1s = (_bsum(jnp.where(lane == r, wlo1, z))
                   + _bsum(jnp.where(lane == r - 16, whi1, z)))
            for j in range(HIDDEN // 16):
                sl = pl.ds(16 * j, 16)
                y0_v[r, sl] = y0_v[r, sl] * w0s + y1_v[r, sl] * w1s
            return 0

        lax.fori_loop(0, CH, row_fn, 0)
        pltpu.sync_copy(y0_v, out_hbm.at[pl.ds(tokbase, CH)])


_c = functools.partial(
    pl.kernel,
    out_type=jax.ShapeDtypeStruct((TOKENS, HIDDEN), jnp.float32),
    mesh=_mesh,
    compiler_params=pltpu.CompilerParams(needs_layout_passes=False),
    scratch_types=[
        pltpu.VMEM((32,), jnp.int32),
        pltpu.VMEM((32,), jnp.int32),
        pltpu.VMEM((32,), jnp.float32),
        pltpu.VMEM((32,), jnp.float32),
        pltpu.VMEM((32, HIDDEN), jnp.float32),
        pltpu.VMEM((32, HIDDEN), jnp.float32),
        pltpu.SemaphoreType.DMA,
        pltpu.SemaphoreType.DMA,
    ],
)(_c_body)


@jax.jit
def kernel(x, router_logits, w13_weight, w2_weight):
    rlt = router_logits.T
    w13t = jnp.transpose(w13_weight, (0, 2, 1)).astype(jnp.bfloat16)
    w2t = jnp.transpose(w2_weight, (0, 2, 1)).astype(jnp.bfloat16)
    ids, wts, counts = _a1(rlt)
    pos, bmeta, xs = _a2(ids, counts, x)
    y = _b_call(bmeta, xs, w13t, w2t)
    out = _c(y, pos, wts)
    return out


# speedup vs baseline: 1.7598x; 1.0294x over previous
import functools

import jax
import jax.numpy as jnp
from jax import lax
from jax.experimental import pallas as pl
from jax.experimental.pallas import tpu as pltpu
from jax.experimental.pallas import tpu_sc as plsc

NUM_EXPERTS = 8
TOP_K = 2
HIDDEN = 1024
INTER = 2048
TOKENS = 2048

NW = 32
TPW = TOKENS // NW
B_R = 256
NB = 16 + NUM_EXPERTS - 1
N_SLOTS = 6144
SPW = N_SLOTS // NW
NEG_INF = float("-inf")

_mesh = plsc.VectorSubcoreMesh(core_axis_name="c", subcore_axis_name="s")


def _wid():
    return lax.axis_index("c") * 16 + lax.axis_index("s")


def _lane():
    return lax.iota(jnp.int32, 16)


def _bsum(x):
    cs = plsc.cumsum(x)
    return plsc.cummax(lax.rev(cs, (0,)))


def _pick(vec, lane_idx):
    return _bsum(jnp.where(_lane() == lane_idx, vec, jnp.zeros_like(vec)))


def _a1_body(rl_hbm, ids_hbm, w_hbm, counts_hbm, rl_v, ids_v, w_v, cnt_v):
    wid = _wid()
    base = wid * TPW
    pltpu.sync_copy(rl_hbm, rl_v)

    cnt = [jnp.zeros((16,), jnp.int32) for _ in range(NUM_EXPERTS)]
    for c in range(TPW // 16):
        ls = [rl_v[e, pl.ds(base + 16 * c, 16)] for e in range(NUM_EXPERTS)]
        best_v = ls[0]
        best_i = jnp.zeros((16,), jnp.int32)
        for e in range(1, NUM_EXPERTS):
            m = ls[e] > best_v
            best_i = jnp.where(m, e, best_i)
            best_v = jnp.where(m, ls[e], best_v)
        sec_v = jnp.full((16,), NEG_INF, jnp.float32)
        sec_i = jnp.zeros((16,), jnp.int32)
        for e in range(NUM_EXPERTS):
            cand = jnp.where(best_i == e, NEG_INF, ls[e])
            m = cand > sec_v
            sec_i = jnp.where(m, e, sec_i)
            sec_v = jnp.where(m, cand, sec_v)
        d = best_v - sec_v
        w2nd = 1.0 / (1.0 + jnp.exp(d))
        w1st = 1.0 - w2nd
        ids_v[0, pl.ds(16 * c, 16)] = best_i
        ids_v[1, pl.ds(16 * c, 16)] = sec_i
        w_v[0, pl.ds(16 * c, 16)] = w1st
        w_v[1, pl.ds(16 * c, 16)] = w2nd
        for e in range(NUM_EXPERTS):
            cnt[e] = cnt[e] + jnp.where(best_i == e, 1, 0)
            cnt[e] = cnt[e] + jnp.where(sec_i == e, 1, 0)

    lane = _lane()
    cvec = jnp.zeros((16,), jnp.int32)
    for e in range(NUM_EXPERTS):
        cvec = jnp.where(lane == e, _bsum(cnt[e]), cvec)
    cnt_v[...] = cvec

    for k in range(TOP_K):
        pltpu.sync_copy(ids_v.at[k], ids_hbm.at[k, pl.ds(base, TPW)])
        pltpu.sync_copy(w_v.at[k], w_hbm.at[k, pl.ds(base, TPW)])
    pltpu.sync_copy(cnt_v, counts_hbm.at[wid])


_a1 = functools.partial(
    pl.kernel,
    out_type=(
        jax.ShapeDtypeStruct((TOP_K, TOKENS), jnp.int32),
        jax.ShapeDtypeStruct((TOP_K, TOKENS), jnp.float32),
        jax.ShapeDtypeStruct((NW, 16), jnp.int32),
    ),
    mesh=_mesh,
    compiler_params=pltpu.CompilerParams(needs_layout_passes=False),
    scratch_types=[
        pltpu.VMEM((NUM_EXPERTS, TOKENS), jnp.float32),
        pltpu.VMEM((TOP_K, TPW), jnp.int32),
        pltpu.VMEM((TOP_K, TPW), jnp.float32),
        pltpu.VMEM((16,), jnp.int32),
    ],
)(_a1_body)


def _global_counts(cnt_v, wid=None):
    total_v = jnp.zeros((16,), jnp.int32)
    for r in range(NW):
        total_v = total_v + cnt_v[r, :]
    s = [_pick(total_v, e) for e in range(NUM_EXPERTS)]
    if wid is None:
        return s, None
    pref_v = lax.fori_loop(0, wid, lambda r, a: a + cnt_v[r, :],
                           jnp.zeros((16,), jnp.int32))
    p = [_pick(pref_v, e) for e in range(NUM_EXPERTS)]
    return s, p


def _block_starts(s):
    bs, cum = [], []
    run_rows = jnp.zeros((16,), jnp.int32)
    run_blocks = jnp.zeros((16,), jnp.int32)
    for e in range(NUM_EXPERTS):
        bs.append(run_rows)
        nb_e = (s[e] + (B_R - 1)) // B_R
        run_blocks = run_blocks + nb_e
        run_rows = run_rows + nb_e * B_R
        cum.append(run_blocks)
    return bs, cum


def _a2_body(ids_hbm, counts_hbm, x_hbm,
             pos_hbm, bmeta_hbm, xs_hbm,
             cnt_v, ids_v, idx0_v, idx1_v, xrows_v, bm_v):
    wid = _wid()
    base = wid * TPW
    pltpu.sync_copy(counts_hbm, cnt_v)
    for k in range(TOP_K):
        pltpu.sync_copy(ids_hbm.at[k, pl.ds(base, TPW)], ids_v.at[k])

    s, p = _global_counts(cnt_v, wid)
    bs, cum = _block_starts(s)
    my_off = [bs[e] + p[e] for e in range(NUM_EXPERTS)]

    for k in range(TOP_K):
        idx_ref = idx0_v if k == 0 else idx1_v
        for c in range(TPW // 16):
            idv = ids_v[k, pl.ds(16 * c, 16)]
            posv = jnp.zeros((16,), jnp.int32)
            for e in range(NUM_EXPERTS):
                m = idv == e
                mi = jnp.where(m, 1, 0)
                csum = plsc.cumsum(mi)
                cand = (my_off[e] - 1) + csum
                posv = jnp.where(m, cand, posv)
                my_off[e] = my_off[e] + plsc.cummax(lax.rev(csum, (0,)))
            idx_ref[pl.ds(16 * c, 16)] = posv

    pltpu.sync_copy(idx0_v, pos_hbm.at[0, pl.ds(base, TPW)])
    pltpu.sync_copy(idx1_v, pos_hbm.at[1, pl.ds(base, TPW)])

    pltpu.sync_copy(x_hbm.at[pl.ds(base, TPW)], xrows_v)
    pltpu.sync_copy(xrows_v, xs_hbm.at[idx0_v])
    pltpu.sync_copy(xrows_v, xs_hbm.at[idx1_v])

    @pl.when(wid == 0)
    def _():
        lane = _lane()
        for v in range(3):
            bvec = lane + 16 * v
            acc = jnp.zeros((16,), jnp.int32)
            for e in range(NUM_EXPERTS):
                acc = acc + jnp.where(bvec >= cum[e], 1, 0)
            bexp = jnp.minimum(acc, NUM_EXPERTS - 1)
            bexp = jnp.where(bvec == NB, cum[NUM_EXPERTS - 1], bexp)
            bm_v[pl.ds(16 * v, 16)] = bexp
        pltpu.sync_copy(bm_v, bmeta_hbm)


_a2 = functools.partial(
    pl.kernel,
    out_type=(
        jax.ShapeDtypeStruct((TOP_K, TOKENS), jnp.int32),
        jax.ShapeDtypeStruct((48,), jnp.int32),
        jax.ShapeDtypeStruct((N_SLOTS, HIDDEN), jnp.float32),
    ),
    mesh=_mesh,
    compiler_params=pltpu.CompilerParams(needs_layout_passes=False),
    scratch_types=[
        pltpu.VMEM((NW, 16), jnp.int32),
        pltpu.VMEM((TOP_K, TPW), jnp.int32),
        pltpu.VMEM((TPW,), jnp.int32),
        pltpu.VMEM((TPW,), jnp.int32),
        pltpu.VMEM((TPW, HIDDEN), jnp.float32),
        pltpu.VMEM((48,), jnp.int32),
    ],
)(_a2_body)


def _b_body(bm_ref, xs_ref, w13_ref, w2_ref, y_ref):
    b = pl.program_id(0)
    nused = bm_ref[NB]

    @pl.when(b < nused)
    def _():
        xb = xs_ref[...].astype(jnp.bfloat16)
        gu = lax.dot_general(xb, w13_ref[0], (((1,), (0,)), ((), ())),
                             preferred_element_type=jnp.float32)
        gate = gu[:, :INTER]
        up = gu[:, INTER:]
        h = ((gate * jax.nn.sigmoid(gate)) * up).astype(jnp.bfloat16)
        y_ref[...] = lax.dot_general(h, w2_ref[0], (((1,), (0,)), ((), ())),
                                     preferred_element_type=jnp.float32)


def _b_call(bmeta, xs2, w13t, w2t):
    def eff(b, sref):
        return jnp.minimum(b, sref[NB] - 1)

    grid_spec = pltpu.PrefetchScalarGridSpec(
        num_scalar_prefetch=1,
        grid=(NB,),
        in_specs=[
            pl.BlockSpec((B_R, HIDDEN), lambda b, sref: (eff(b, sref), 0)),
            pl.BlockSpec((1, HIDDEN, 2 * INTER),
                         lambda b, sref: (sref[eff(b, sref)], 0, 0)),
            pl.BlockSpec((1, INTER, HIDDEN),
                         lambda b, sref: (sref[eff(b, sref)], 0, 0)),
        ],
        out_specs=pl.BlockSpec((B_R, HIDDEN),
                               lambda b, sref: (eff(b, sref), 0)),
    )
    return pl.pallas_call(
        _b_body,
        grid_spec=grid_spec,
        out_shape=jax.ShapeDtypeStruct((NB * B_R, HIDDEN), jnp.float32),
        compiler_params=pltpu.CompilerParams(
            dimension_semantics=("arbitrary",),
        ),
    )(bmeta, xs2, w13t, w2t)


def _c_body(y_hbm, pos_hbm, w_hbm, out_hbm,
            idx0_v, idx1_v, w0_v, w1_v, y0_v, y1_v, sem_a, sem_b):
    wid = _wid()
    lane = _lane()
    CH = 32
    for cc in range(TPW // CH):
        tokbase = wid * TPW + cc * CH
        pltpu.sync_copy(pos_hbm.at[0, pl.ds(tokbase, CH)], idx0_v)
        pltpu.sync_copy(pos_hbm.at[1, pl.ds(tokbase, CH)], idx1_v)
        pltpu.sync_copy(w_hbm.at[0, pl.ds(tokbase, CH)], w0_v)
        pltpu.sync_copy(w_hbm.at[1, pl.ds(tokbase, CH)], w1_v)
        cp_a = pltpu.async_copy(y_hbm.at[idx0_v], y0_v, sem_a)
        cp_b = pltpu.async_copy(y_hbm.at[idx1_v], y1_v, sem_b)
        cp_a.wait()
        cp_b.wait()

        wlo0, whi0 = w0_v[pl.ds(0, 16)], w0_v[pl.ds(16, 16)]
        wlo1, whi1 = w1_v[pl.ds(0, 16)], w1_v[pl.ds(16, 16)]

        def row_fn(r, _):
            z = jnp.zeros((16,), jnp.float32)
            w0s = (_bsum(jnp.where(lane == r, wlo0, z))
                   + _bsum(jnp.where(lane == r - 16, whi0, z)))
            w1s = (_bsum(jnp.where(lane == r, wlo1, z))
                   + _bsum(jnp.where(lane == r - 16, whi1, z)))
            for j in range(HIDDEN // 16):
                sl = pl.ds(16 * j, 16)
                y0_v[r, sl] = y0_v[r, sl] * w0s + y1_v[r, sl] * w1s
            return 0

        lax.fori_loop(0, CH, row_fn, 0)
        pltpu.sync_copy(y0_v, out_hbm.at[pl.ds(tokbase, CH)])


_c = functools.partial(
    pl.kernel,
    out_type=jax.ShapeDtypeStruct((TOKENS, HIDDEN), jnp.float32),
    mesh=_mesh,
    compiler_params=pltpu.CompilerParams(needs_layout_passes=False),
    scratch_types=[
        pltpu.VMEM((32,), jnp.int32),
        pltpu.VMEM((32,), jnp.int32),
        pltpu.VMEM((32,), jnp.float32),
        pltpu.VMEM((32,), jnp.float32),
        pltpu.VMEM((32, HIDDEN), jnp.float32),
        pltpu.VMEM((32, HIDDEN), jnp.float32),
        pltpu.SemaphoreType.DMA,
        pltpu.SemaphoreType.DMA,
    ],
)(_c_body)


@jax.jit
def kernel(x, router_logits, w13_weight, w2_weight):
    rlt = router_logits.T
    w13t = jnp.transpose(w13_weight, (0, 2, 1)).astype(jnp.bfloat16)
    w2t = jnp.transpose(w2_weight, (0, 2, 1)).astype(jnp.bfloat16)
    ids, wts, counts = _a1(rlt)
    pos, bmeta, xs = _a2(ids, counts, x)
    y = _b_call(bmeta, xs, w13t, w2t)
    out = _c(y, pos, wts)
    return out
